# baseline (speedup 1.0000x reference)
import functools

import jax
import jax.numpy as jnp
from jax import lax
from jax.experimental import pallas as pl
from jax.experimental.pallas import tpu as pltpu
from jax.experimental.pallas import tpu_sc as plsc

_N = 10000
_E = 320000
_D = 128
_R = 8

_NC = 2
_NS = 16
_NT = _NC * _NS
_NPASS = _D // _NT
_SEC = 3200
_NSECS = _E // _SEC
_VPS = _SEC // 16
_ZW = _R * _N

_BN = 400
_NB = _N // _BN


def _sidx_body(et_ref, dst_ref, src_ref, o_ref):
    o_ref[...] = ((et_ref[...] * _N + dst_ref[...]) << 14) | src_ref[...]


def _sidx(et2d, dst2d, src2d):
    rows = et2d.shape[0]
    return pl.pallas_call(
        _sidx_body,
        grid=(1,),
        in_specs=[
            pl.BlockSpec((rows, 128), lambda i: (0, 0)),
            pl.BlockSpec((rows, 128), lambda i: (0, 0)),
            pl.BlockSpec((rows, 128), lambda i: (0, 0)),
        ],
        out_specs=pl.BlockSpec((rows, 128), lambda i: (0, 0)),
        out_shape=jax.ShapeDtypeStruct((rows, 128), jnp.int32),
    )(et2d, dst2d, src2d)


def _tr_body(x_ref, o_ref):
    o_ref[...] = x_ref[...].T


def _transpose(x):
    return pl.pallas_call(
        _tr_body,
        grid=(1,),
        in_specs=[pl.BlockSpec((_N, _D), lambda i: (0, 0))],
        out_specs=pl.BlockSpec((_D, _N), lambda i: (0, 0)),
        out_shape=jax.ShapeDtypeStruct((_D, _N), jnp.float32),
    )(x)


def _make_sc_zagg():
    mesh = plsc.VectorSubcoreMesh(core_axis_name="c", subcore_axis_name="s")

    @functools.partial(
        pl.kernel,
        mesh=mesh,
        compiler_params=pltpu.CompilerParams(needs_layout_passes=False),
        out_type=jax.ShapeDtypeStruct((_NPASS * _NT * _ZW,), jnp.float32),
        scratch_types=[
            pltpu.VMEM((_N,), jnp.float32),
            pltpu.VMEM((_ZW,), jnp.float32),
            [pltpu.VMEM((_SEC,), jnp.int32)] * 2,
            pltpu.SemaphoreType.DMA,
            pltpu.SemaphoreType.DMA,
            [pltpu.SemaphoreType.DMA] * 2,
        ],
    )
    def sc_zagg(xt_hbm, ep_hbm, zeros_hbm, out_hbm,
                xcol_v, z_v, ep_v, semx, semz, esems):
        c = lax.axis_index("c")
        s = lax.axis_index("s")
        w = s * _NC + c

        def fetch_sec(k, b):
            pltpu.async_copy(ep_hbm.at[pl.ds(k * _SEC, _SEC)], ep_v[b],
                             esems[b])

        def wait_sec(k, b):
            pltpu.make_async_copy(ep_hbm.at[pl.ds(k * _SEC, _SEC)],
                                  ep_v[b], esems[b]).wait()

        def do_pass(p, carry):
            col = p * _NT + w
            cx = pltpu.async_copy(xt_hbm.at[pl.ds(col * _N, _N)], xcol_v,
                                  semx)
            cz = pltpu.async_copy(zeros_hbm, z_v, semz)
            fetch_sec(0, 0)
            cx.wait()
            cz.wait()

            def run_sec(k, b):
                @pl.when(k + 1 < _NSECS)
                def _():
                    fetch_sec(k + 1, 1 - b)

                wait_sec(k, b)

                for v in range(_VPS):
                    ev = ep_v[b][pl.ds(v * 16, 16)]
                    sv = ev & 16383
                    iv = lax.shift_right_logical(ev, 14)
                    vals = plsc.load_gather(xcol_v, [sv])
                    plsc.addupdate_scatter(z_v, [iv], vals)

            def section2(k2, carry2):
                run_sec(2 * k2, 0)
                run_sec(2 * k2 + 1, 1)
                return carry2

            lax.fori_loop(0, _NSECS // 2, section2, 0)
            for r_ in range(_R):
                pltpu.sync_copy(z_v.at[pl.ds(r_ * _N, _N)],
                                out_hbm.at[pl.ds(r_ * _D * _N + col * _N, _N)])
            return carry

        lax.fori_loop(0, _NPASS, do_pass, 0)

    return sc_zagg


_sc_zagg = _make_sc_zagg()


def _dotT(zblk, wblk):
    return lax.dot_general(zblk, wblk, (((0,), (0,)), ((), ())),
                           preferred_element_type=jnp.float32)


def _combine1_body(z_ref, x_ref, w_ref, b_ref, oh_ref, oht_ref, acc_ref):
    r = pl.program_id(0)

    @pl.when(r == 0)
    def _():
        acc_ref[...] = jnp.zeros_like(acc_ref)

    @pl.when(r < _R)
    def _():
        acc_ref[...] += _dotT(z_ref[0], w_ref[0])

    @pl.when(r == _R)
    def _():
        h = jnp.maximum(
            acc_ref[...]
            + jnp.dot(x_ref[...], w_ref[0], preferred_element_type=jnp.float32)
            + b_ref[...], 0.0)
        oh_ref[...] = h
        oht_ref[...] = h.T


def _combine1(z, x, wall, b):
    return pl.pallas_call(
        _combine1_body,
        grid=(_R + 1,),
        in_specs=[
            pl.BlockSpec((1, _D, _N), lambda r: (jnp.minimum(r, _R - 1), 0, 0)),
            pl.BlockSpec((_N, _D), lambda r: (0, 0)),
            pl.BlockSpec((1, _D, _D), lambda r: (r, 0, 0)),
            pl.BlockSpec((1, _D), lambda r: (0, 0)),
        ],
        out_specs=[
            pl.BlockSpec((_N, _D), lambda r: (0, 0)),
            pl.BlockSpec((_D, _N), lambda r: (0, 0)),
        ],
        out_shape=[
            jax.ShapeDtypeStruct((_N, _D), jnp.float32),
            jax.ShapeDtypeStruct((_D, _N), jnp.float32),
        ],
        scratch_shapes=[pltpu.VMEM((_N, _D), jnp.float32)],
    )(z, x, wall, b)


def _combine2_body(z_ref, x_ref, w_ref, b_ref, fcw_ref, fcb_ref, o_ref,
                   acc_ref):
    r = pl.program_id(0)

    @pl.when(r == 0)
    def _():
        acc_ref[...] = jnp.zeros_like(acc_ref)

    @pl.when(r < _R)
    def _():
        acc_ref[...] += _dotT(z_ref[0], w_ref[0])

    @pl.when(r == _R)
    def _():
        h = jnp.maximum(
            acc_ref[...]
            + jnp.dot(x_ref[...], w_ref[0], preferred_element_type=jnp.float32)
            + b_ref[...], 0.0)
        hg = jnp.sum(h, axis=0, keepdims=True) * (1.0 / _N)
        zz = jnp.sum(hg * fcw_ref[...], keepdims=True) + fcb_ref[...]
        o_ref[...] = 1.0 / (1.0 + jnp.exp(-zz))


def _combine2(z, x, wall, b, fcw_row, fcb):
    return pl.pallas_call(
        _combine2_body,
        grid=(_R + 1,),
        in_specs=[
            pl.BlockSpec((1, _D, _N), lambda r: (jnp.minimum(r, _R - 1), 0, 0)),
            pl.BlockSpec((_N, _D), lambda r: (0, 0)),
            pl.BlockSpec((1, _D, _D), lambda r: (r, 0, 0)),
            pl.BlockSpec((1, _D), lambda r: (0, 0)),
            pl.BlockSpec((1, _D), lambda r: (0, 0)),
            pl.BlockSpec((1, 1), lambda r: (0, 0)),
        ],
        out_specs=pl.BlockSpec((1, 1), lambda r: (0, 0)),
        out_shape=jax.ShapeDtypeStruct((1, 1), jnp.float32),
        scratch_shapes=[pltpu.VMEM((_N, _D), jnp.float32)],
    )(z, x, wall, b, fcw_row, fcb)


def kernel(in_feat, edge_index, e_types, W1, Wself1, b1, W2, Wself2, b2,
           fc_w, fc_b):
    src = edge_index[0]
    dst = edge_index[1]

    ep = _sidx(e_types.reshape(-1, 128), dst.reshape(-1, 128),
               src.reshape(-1, 128)).reshape(_E)
    zeros = jnp.zeros((_ZW,), jnp.float32)
    wall1 = jnp.concatenate([W1, Wself1[None]], axis=0)
    wall2 = jnp.concatenate([W2, Wself2[None]], axis=0)

    xt0 = _transpose(in_feat).reshape(_D * _N)
    z1 = _sc_zagg(xt0, ep, zeros).reshape(_R, _D, _N)
    h1, h1t = _combine1(z1, in_feat, wall1, b1.reshape(1, _D))
    z2 = _sc_zagg(h1t.reshape(_D * _N), ep, zeros).reshape(_R, _D, _N)
    return _combine2(z2, h1, wall2, b2.reshape(1, _D), fc_w.reshape(1, _D),
                     fc_b.reshape(1, 1))


# speedup vs baseline: 1.6230x; 1.6230x over previous
import functools

import jax
import jax.numpy as jnp
from jax import lax
from jax.experimental import pallas as pl
from jax.experimental.pallas import tpu as pltpu
from jax.experimental.pallas import tpu_sc as plsc

_N = 10000
_E = 320000
_D = 128
_R = 8

_NC = 2
_NS = 16
_NT = _NC * _NS
_CH = 128
_NCHUNK = 80
_SECN = 16
_NSEC = _NCHUNK // _SECN
_EPT = _CH * _NCHUNK
_EPAD = _NT * _EPT
_NPAD = 10240
_RPT = _NPAD // _NS

_BN = 400
_NB = _N // _BN


def _xw_body(x_ref, w_ref, o_ref):
    o_ref[0] = jnp.dot(x_ref[...], w_ref[0], preferred_element_type=jnp.float32)


def _xw(x, W):
    return pl.pallas_call(
        _xw_body,
        grid=(_NB, _R),
        in_specs=[
            pl.BlockSpec((_BN, _D), lambda i, r: (i, 0)),
            pl.BlockSpec((1, _D, _D), lambda i, r: (r, 0, 0)),
        ],
        out_specs=pl.BlockSpec((1, _BN, _D), lambda i, r: (r, i, 0)),
        out_shape=jax.ShapeDtypeStruct((_R, _N, _D), jnp.float32),
    )(x, W)


def _selfp_body(x_ref, w_ref, o_ref):
    o_ref[...] = jnp.dot(x_ref[...], w_ref[...], preferred_element_type=jnp.float32)


def _selfp(x, Wself):
    return pl.pallas_call(
        _selfp_body,
        grid=(_NB,),
        in_specs=[
            pl.BlockSpec((_BN, _D), lambda i: (i, 0)),
            pl.BlockSpec((_D, _D), lambda i: (0, 0)),
        ],
        out_specs=pl.BlockSpec((_BN, _D), lambda i: (i, 0)),
        out_shape=jax.ShapeDtypeStruct((_N, _D), jnp.float32),
    )(x, Wself)


def _gidx_body(et_ref, src_ref, o_ref):
    o_ref[...] = et_ref[...] * _N + src_ref[...]


def _gidx(et2d, src2d):
    rows = et2d.shape[0]
    return pl.pallas_call(
        _gidx_body,
        grid=(2,),
        in_specs=[
            pl.BlockSpec((rows // 2, _CH), lambda i: (i, 0)),
            pl.BlockSpec((rows // 2, _CH), lambda i: (i, 0)),
        ],
        out_specs=pl.BlockSpec((rows // 2, _CH), lambda i: (i, 0)),
        out_shape=jax.ShapeDtypeStruct((rows, _CH), jnp.int32),
    )(et2d, src2d)


def _make_sc_agg():
    mesh = plsc.VectorSubcoreMesh(core_axis_name="c", subcore_axis_name="s")

    @functools.partial(
        pl.kernel,
        mesh=mesh,
        out_type=jax.ShapeDtypeStruct((_NC, _NS, _RPT, _D), jnp.float32),
        scratch_types=[
            pltpu.VMEM((_SECN, _CH), jnp.int32),
            pltpu.VMEM((_SECN, _CH), jnp.int32),
            pltpu.VMEM((2, _CH, _D), jnp.float32),
            pltpu.VMEM_SHARED((_NPAD, _D), jnp.float32),
            pltpu.SemaphoreType.DMA,
            pltpu.SemaphoreType.DMA,
        ],
    )
    def sc_agg(xw_hbm, gidx_hbm, didx_hbm, zeros_hbm, out_hbm,
               gidx_v, didx_v, rows_v, agg_sh, sem0, sem1):
        c = lax.axis_index("c")
        s = lax.axis_index("s")
        row0 = s * _RPT
        pltpu.sync_copy(zeros_hbm.at[pl.ds(row0, _RPT)],
                        agg_sh.at[pl.ds(row0, _RPT)])
        plsc.subcore_barrier()
        sems = (sem0, sem1)

        def section(k, carry):
            pltpu.sync_copy(gidx_hbm.at[c, s, k], gidx_v)
            pltpu.sync_copy(didx_hbm.at[c, s, k], didx_v)
            pltpu.async_copy(xw_hbm.at[gidx_v.at[0]], rows_v.at[0], sems[0])
            for j in range(_SECN):
                b = j % 2
                if j + 1 < _SECN:
                    nb = (j + 1) % 2
                    pltpu.async_copy(xw_hbm.at[gidx_v.at[j + 1]],
                                     rows_v.at[nb], sems[nb])
                pltpu.make_async_copy(xw_hbm.at[gidx_v.at[j]],
                                      rows_v.at[b], sems[b]).wait()
                pltpu.sync_copy(rows_v.at[b], agg_sh.at[didx_v.at[j]],
                                add=True)
            return carry

        lax.fori_loop(0, _NSEC, section, 0)
        plsc.subcore_barrier()
        pltpu.sync_copy(agg_sh.at[pl.ds(row0, _RPT)], out_hbm.at[c, s])

    return sc_agg


_sc_agg = _make_sc_agg()


def _combine1_body(p_ref, sp_ref, b_ref, o_ref):
    o_ref[...] = jnp.maximum(
        p_ref[0] + p_ref[1] + sp_ref[...] + b_ref[...], 0.0)


def _combine1(p, sp, b):
    return pl.pallas_call(
        _combine1_body,
        grid=(_NB,),
        in_specs=[
            pl.BlockSpec((2, _BN, _D), lambda i: (0, i, 0)),
            pl.BlockSpec((_BN, _D), lambda i: (i, 0)),
            pl.BlockSpec((1, _D), lambda i: (0, 0)),
        ],
        out_specs=pl.BlockSpec((_BN, _D), lambda i: (i, 0)),
        out_shape=jax.ShapeDtypeStruct((_N, _D), jnp.float32),
    )(p, sp, b)


def _combine2_body(p_ref, sp_ref, b_ref, fcw_ref, fcb_ref, o_ref, acc_ref):
    i = pl.program_id(0)

    @pl.when(i == 0)
    def _():
        acc_ref[...] = jnp.zeros_like(acc_ref)

    h = jnp.maximum(p_ref[0] + p_ref[1] + sp_ref[...] + b_ref[...], 0.0)
    acc_ref[0:1] += jnp.sum(h, axis=0, keepdims=True)

    @pl.when(i == pl.num_programs(0) - 1)
    def _():
        hg = acc_ref[0:1] * (1.0 / _N)
        z = jnp.sum(hg * fcw_ref[...], keepdims=True) + fcb_ref[...]
        o_ref[...] = 1.0 / (1.0 + jnp.exp(-z))


def _combine2(p, sp, b, fcw_row, fcb):
    return pl.pallas_call(
        _combine2_body,
        grid=(_NB,),
        in_specs=[
            pl.BlockSpec((2, _BN, _D), lambda i: (0, i, 0)),
            pl.BlockSpec((_BN, _D), lambda i: (i, 0)),
            pl.BlockSpec((1, _D), lambda i: (0, 0)),
            pl.BlockSpec((1, _D), lambda i: (0, 0)),
            pl.BlockSpec((1, 1), lambda i: (0, 0)),
        ],
        out_specs=pl.BlockSpec((1, 1), lambda i: (0, 0)),
        out_shape=jax.ShapeDtypeStruct((1, 1), jnp.float32),
        scratch_shapes=[pltpu.VMEM((8, _D), jnp.float32)],
    )(p, sp, b, fcw_row, fcb)


def kernel(in_feat, edge_index, e_types, W1, Wself1, b1, W2, Wself2, b2,
           fc_w, fc_b):
    src = edge_index[0]
    dst = edge_index[1]
    pad = _EPAD - _E
    et_p = jnp.concatenate([e_types, jnp.zeros((pad,), jnp.int32)])
    src_p = jnp.concatenate([src, jnp.zeros((pad,), jnp.int32)])
    dst_p = jnp.concatenate([dst, jnp.full((pad,), _N, jnp.int32)])

    gidx = _gidx(et_p.reshape(-1, _CH), src_p.reshape(-1, _CH))
    gidx4 = gidx.reshape(_NC, _NS, _NSEC, _SECN, _CH)
    didx4 = dst_p.reshape(_NC, _NS, _NSEC, _SECN, _CH)
    zeros = jnp.zeros((_NPAD, _D), jnp.float32)

    def layer(x, W, Wself):
        xw = _xw(x, W)
        sp = _selfp(x, Wself)
        p = _sc_agg(xw.reshape(_R * _N, _D), gidx4, didx4, zeros)
        return p.reshape(_NC, _NPAD, _D), sp

    p1, sp1 = layer(in_feat, W1, Wself1)
    h1 = _combine1(p1, sp1, b1.reshape(1, _D))
    p2, sp2 = layer(h1, W2, Wself2)
    return _combine2(p2, sp2, b2.reshape(1, _D), fc_w.reshape(1, _D),
                     fc_b.reshape(1, 1))


# speedup vs baseline: 1.6330x; 1.0062x over previous
import functools

import jax
import jax.numpy as jnp
from jax import lax
from jax.experimental import pallas as pl
from jax.experimental.pallas import tpu as pltpu
from jax.experimental.pallas import tpu_sc as plsc

_N = 10000
_E = 320000
_D = 128
_R = 8

_NC = 2
_NS = 16
_NT = _NC * _NS
_CH = 128
_NCHUNK = 80
_SECN = 16
_NSEC = _NCHUNK // _SECN
_EPT = _CH * _NCHUNK
_EPAD = _NT * _EPT
_NPAD = 10240
_RPT = _NPAD // _NS

_BN = 400
_NB = _N // _BN


def _xw_body(x_ref, w_ref, o_ref):
    o_ref[0] = jnp.dot(x_ref[...], w_ref[0], preferred_element_type=jnp.float32)


def _xw(x, W):
    return pl.pallas_call(
        _xw_body,
        grid=(_NB, _R),
        in_specs=[
            pl.BlockSpec((_BN, _D), lambda i, r: (i, 0)),
            pl.BlockSpec((1, _D, _D), lambda i, r: (r, 0, 0)),
        ],
        out_specs=pl.BlockSpec((1, _BN, _D), lambda i, r: (r, i, 0)),
        out_shape=jax.ShapeDtypeStruct((_R, _N, _D), jnp.float32),
    )(x, W)


def _selfp_body(x_ref, w_ref, o_ref):
    o_ref[...] = jnp.dot(x_ref[...], w_ref[...], preferred_element_type=jnp.float32)


def _selfp(x, Wself):
    return pl.pallas_call(
        _selfp_body,
        grid=(_NB,),
        in_specs=[
            pl.BlockSpec((_BN, _D), lambda i: (i, 0)),
            pl.BlockSpec((_D, _D), lambda i: (0, 0)),
        ],
        out_specs=pl.BlockSpec((_BN, _D), lambda i: (i, 0)),
        out_shape=jax.ShapeDtypeStruct((_N, _D), jnp.float32),
    )(x, Wself)


def _gidx_body(et_ref, src_ref, o_ref):
    o_ref[...] = et_ref[...] * _N + src_ref[...]


def _gidx(et2d, src2d):
    rows = et2d.shape[0]
    return pl.pallas_call(
        _gidx_body,
        grid=(2,),
        in_specs=[
            pl.BlockSpec((rows // 2, _CH), lambda i: (i, 0)),
            pl.BlockSpec((rows // 2, _CH), lambda i: (i, 0)),
        ],
        out_specs=pl.BlockSpec((rows // 2, _CH), lambda i: (i, 0)),
        out_shape=jax.ShapeDtypeStruct((rows, _CH), jnp.int32),
    )(et2d, src2d)


def _make_sc_agg():
    mesh = plsc.VectorSubcoreMesh(core_axis_name="c", subcore_axis_name="s")

    @functools.partial(
        pl.kernel,
        mesh=mesh,
        out_type=jax.ShapeDtypeStruct((_NC, _NS, _RPT, _D), jnp.float32),
        scratch_types=[
            [pltpu.VMEM((_SECN, _CH), jnp.int32)] * 2,
            [pltpu.VMEM((_SECN, _CH), jnp.int32)] * 2,
            pltpu.VMEM((2, _CH, _D), jnp.float32),
            pltpu.VMEM_SHARED((_NPAD, _D), jnp.float32),
            pltpu.SemaphoreType.DMA,
            pltpu.SemaphoreType.DMA,
            pltpu.SemaphoreType.DMA,
            [pltpu.SemaphoreType.DMA] * 2,
        ],
    )
    def sc_agg(xw_hbm, gidx_hbm, didx_hbm, zeros_hbm, out_hbm,
               gidx_v, didx_v, rows_v, agg_sh, sem0, sem1, semz, isems):
        c = lax.axis_index("c")
        s = lax.axis_index("s")
        row0 = s * _RPT
        sems = (sem0, sem1)

        def fetch_idx(k, ib):
            pltpu.async_copy(gidx_hbm.at[c, s, k], gidx_v[ib], isems[ib])
            pltpu.async_copy(didx_hbm.at[c, s, k], didx_v[ib], isems[ib])

        def wait_idx(k, ib):
            pltpu.make_async_copy(gidx_hbm.at[c, s, k], gidx_v[ib],
                                  isems[ib]).wait()
            pltpu.make_async_copy(didx_hbm.at[c, s, k], didx_v[ib],
                                  isems[ib]).wait()

        cz = pltpu.async_copy(zeros_hbm.at[pl.ds(row0, _RPT)],
                              agg_sh.at[pl.ds(row0, _RPT)], semz)
        fetch_idx(0, 0)
        wait_idx(0, 0)
        cz.wait()
        plsc.subcore_barrier()

        for k in range(_NSEC):
            ib = k % 2
            if k + 1 < _NSEC:
                fetch_idx(k + 1, 1 - ib)
            if k > 0:
                wait_idx(k, ib)
            gv = gidx_v[ib]
            dv = didx_v[ib]
            pltpu.async_copy(xw_hbm.at[gv.at[0]], rows_v.at[0], sems[0])
            for j in range(_SECN):
                b = j % 2
                if j + 1 < _SECN:
                    nb = (j + 1) % 2
                    pltpu.async_copy(xw_hbm.at[gv.at[j + 1]],
                                     rows_v.at[nb], sems[nb])
                pltpu.make_async_copy(xw_hbm.at[gv.at[j]],
                                      rows_v.at[b], sems[b]).wait()
                pltpu.sync_copy(rows_v.at[b], agg_sh.at[dv.at[j]],
                                add=True)
        plsc.subcore_barrier()
        pltpu.sync_copy(agg_sh.at[pl.ds(row0, _RPT)], out_hbm.at[c, s])

    return sc_agg


_sc_agg = _make_sc_agg()


def _combine1_body(p_ref, sp_ref, b_ref, o_ref):
    o_ref[...] = jnp.maximum(
        p_ref[0] + p_ref[1] + sp_ref[...] + b_ref[...], 0.0)


def _combine1(p, sp, b):
    return pl.pallas_call(
        _combine1_body,
        grid=(_NB,),
        in_specs=[
            pl.BlockSpec((2, _BN, _D), lambda i: (0, i, 0)),
            pl.BlockSpec((_BN, _D), lambda i: (i, 0)),
            pl.BlockSpec((1, _D), lambda i: (0, 0)),
        ],
        out_specs=pl.BlockSpec((_BN, _D), lambda i: (i, 0)),
        out_shape=jax.ShapeDtypeStruct((_N, _D), jnp.float32),
    )(p, sp, b)


def _combine2_body(p_ref, sp_ref, b_ref, fcw_ref, fcb_ref, o_ref, acc_ref):
    i = pl.program_id(0)

    @pl.when(i == 0)
    def _():
        acc_ref[...] = jnp.zeros_like(acc_ref)

    h = jnp.maximum(p_ref[0] + p_ref[1] + sp_ref[...] + b_ref[...], 0.0)
    acc_ref[0:1] += jnp.sum(h, axis=0, keepdims=True)

    @pl.when(i == pl.num_programs(0) - 1)
    def _():
        hg = acc_ref[0:1] * (1.0 / _N)
        z = jnp.sum(hg * fcw_ref[...], keepdims=True) + fcb_ref[...]
        o_ref[...] = 1.0 / (1.0 + jnp.exp(-z))


def _combine2(p, sp, b, fcw_row, fcb):
    return pl.pallas_call(
        _combine2_body,
        grid=(_NB,),
        in_specs=[
            pl.BlockSpec((2, _BN, _D), lambda i: (0, i, 0)),
            pl.BlockSpec((_BN, _D), lambda i: (i, 0)),
            pl.BlockSpec((1, _D), lambda i: (0, 0)),
            pl.BlockSpec((1, _D), lambda i: (0, 0)),
            pl.BlockSpec((1, 1), lambda i: (0, 0)),
        ],
        out_specs=pl.BlockSpec((1, 1), lambda i: (0, 0)),
        out_shape=jax.ShapeDtypeStruct((1, 1), jnp.float32),
        scratch_shapes=[pltpu.VMEM((8, _D), jnp.float32)],
    )(p, sp, b, fcw_row, fcb)


def kernel(in_feat, edge_index, e_types, W1, Wself1, b1, W2, Wself2, b2,
           fc_w, fc_b):
    src = edge_index[0]
    dst = edge_index[1]
    pad = _EPAD - _E
    et_p = jnp.concatenate([e_types, jnp.zeros((pad,), jnp.int32)])
    src_p = jnp.concatenate([src, jnp.zeros((pad,), jnp.int32)])
    dst_p = jnp.concatenate([dst, jnp.full((pad,), _N, jnp.int32)])

    gidx = _gidx(et_p.reshape(-1, _CH), src_p.reshape(-1, _CH))
    gidx4 = gidx.reshape(_NC, _NS, _NSEC, _SECN, _CH)
    didx4 = dst_p.reshape(_NC, _NS, _NSEC, _SECN, _CH)
    zeros = jnp.zeros((_NPAD, _D), jnp.float32)

    def layer(x, W, Wself):
        xw = _xw(x, W)
        sp = _selfp(x, Wself)
        p = _sc_agg(xw.reshape(_R * _N, _D), gidx4, didx4, zeros)
        return p.reshape(_NC, _NPAD, _D), sp

    p1, sp1 = layer(in_feat, W1, Wself1)
    h1 = _combine1(p1, sp1, b1.reshape(1, _D))
    p2, sp2 = layer(h1, W2, Wself2)
    return _combine2(p2, sp2, b2.reshape(1, _D), fc_w.reshape(1, _D),
                     fc_b.reshape(1, 1))


# speedup vs baseline: 1.6486x; 1.0095x over previous
import functools

import jax
import jax.numpy as jnp
from jax import lax
from jax.experimental import pallas as pl
from jax.experimental.pallas import tpu as pltpu
from jax.experimental.pallas import tpu_sc as plsc

_N = 10000
_E = 320000
_D = 128
_R = 8

_NC = 2
_NS = 16
_NT = _NC * _NS
_CH = 128
_NCHUNK = 80
_SECN = 16
_NSEC = _NCHUNK // _SECN
_EPT = _CH * _NCHUNK
_EPAD = _NT * _EPT
_NPAD = 10240
_RPT = _NPAD // _NS

_BN = 400
_NB = _N // _BN


def _xw_body(x_ref, w_ref, o_ref):
    o_ref[0] = jnp.dot(x_ref[...], w_ref[0], preferred_element_type=jnp.float32)


def _xw(x, W):
    return pl.pallas_call(
        _xw_body,
        grid=(_NB, _R),
        in_specs=[
            pl.BlockSpec((_BN, _D), lambda i, r: (i, 0)),
            pl.BlockSpec((1, _D, _D), lambda i, r: (r, 0, 0)),
        ],
        out_specs=pl.BlockSpec((1, _BN, _D), lambda i, r: (r, i, 0)),
        out_shape=jax.ShapeDtypeStruct((_R, _N, _D), jnp.float32),
    )(x, W)


def _selfp_body(x_ref, w_ref, o_ref):
    o_ref[...] = jnp.dot(x_ref[...], w_ref[...], preferred_element_type=jnp.float32)


def _selfp(x, Wself):
    return pl.pallas_call(
        _selfp_body,
        grid=(_NB,),
        in_specs=[
            pl.BlockSpec((_BN, _D), lambda i: (i, 0)),
            pl.BlockSpec((_D, _D), lambda i: (0, 0)),
        ],
        out_specs=pl.BlockSpec((_BN, _D), lambda i: (i, 0)),
        out_shape=jax.ShapeDtypeStruct((_N, _D), jnp.float32),
    )(x, Wself)


def _gidx_body(et_ref, src_ref, o_ref):
    o_ref[...] = et_ref[...] * _N + src_ref[...]


def _gidx(et2d, src2d):
    rows = et2d.shape[0]
    return pl.pallas_call(
        _gidx_body,
        grid=(2,),
        in_specs=[
            pl.BlockSpec((rows // 2, _CH), lambda i: (i, 0)),
            pl.BlockSpec((rows // 2, _CH), lambda i: (i, 0)),
        ],
        out_specs=pl.BlockSpec((rows // 2, _CH), lambda i: (i, 0)),
        out_shape=jax.ShapeDtypeStruct((rows, _CH), jnp.int32),
    )(et2d, src2d)


def _make_sc_agg():
    mesh = plsc.VectorSubcoreMesh(core_axis_name="c", subcore_axis_name="s")

    @functools.partial(
        pl.kernel,
        mesh=mesh,
        out_type=jax.ShapeDtypeStruct((_NC, _NS, _RPT, _D), jnp.float32),
        scratch_types=[
            [pltpu.VMEM((_SECN, _CH), jnp.int32)] * 2,
            [pltpu.VMEM((_SECN, _CH), jnp.int32)] * 2,
            pltpu.VMEM((2, _CH, _D), jnp.float32),
            pltpu.VMEM_SHARED((_NPAD, _D), jnp.float32),
            pltpu.SemaphoreType.DMA,
            pltpu.SemaphoreType.DMA,
            pltpu.SemaphoreType.DMA,
            [pltpu.SemaphoreType.DMA] * 2,
        ],
    )
    def sc_agg(xw_hbm, gidx_hbm, didx_hbm, zeros_hbm, out_hbm,
               gidx_v, didx_v, rows_v, agg_sh, sem0, sem1, semz, isems):
        c = lax.axis_index("c")
        s = lax.axis_index("s")
        row0 = s * _RPT
        sems = (sem0, sem1)

        def fetch_idx(k, ib):
            pltpu.async_copy(gidx_hbm.at[c, s, k], gidx_v[ib], isems[ib])
            pltpu.async_copy(didx_hbm.at[c, s, k], didx_v[ib], isems[ib])

        def wait_idx(k, ib):
            pltpu.make_async_copy(gidx_hbm.at[c, s, k], gidx_v[ib],
                                  isems[ib]).wait()
            pltpu.make_async_copy(didx_hbm.at[c, s, k], didx_v[ib],
                                  isems[ib]).wait()

        cz = pltpu.async_copy(zeros_hbm.at[pl.ds(row0, _RPT)],
                              agg_sh.at[pl.ds(row0, _RPT)], semz)
        fetch_idx(0, 0)
        wait_idx(0, 0)
        cz.wait()
        plsc.subcore_barrier()

        fetch_idx(1, 1)
        pltpu.async_copy(xw_hbm.at[gidx_v[0].at[0]], rows_v.at[0], sems[0])
        for g in range(_NCHUNK):
            ib = (g // _SECN) % 2
            b = g % 2
            if g % _SECN == 0 and 0 < g and g // _SECN + 1 < _NSEC:
                fetch_idx(g // _SECN + 1, 1 - ib)
            jn = g + 1
            if jn < _NCHUNK:
                nib = (jn // _SECN) % 2
                if jn % _SECN == 0:
                    wait_idx(jn // _SECN, nib)
                pltpu.async_copy(xw_hbm.at[gidx_v[nib].at[jn % _SECN]],
                                 rows_v.at[jn % 2], sems[jn % 2])
            pltpu.make_async_copy(xw_hbm.at[gidx_v[ib].at[g % _SECN]],
                                  rows_v.at[b], sems[b]).wait()
            pltpu.sync_copy(rows_v.at[b],
                            agg_sh.at[didx_v[ib].at[g % _SECN]], add=True)
        plsc.subcore_barrier()
        pltpu.sync_copy(agg_sh.at[pl.ds(row0, _RPT)], out_hbm.at[c, s])

    return sc_agg


_sc_agg = _make_sc_agg()


def _combine1_body(p_ref, sp_ref, b_ref, o_ref):
    o_ref[...] = jnp.maximum(
        p_ref[0] + p_ref[1] + sp_ref[...] + b_ref[...], 0.0)


def _combine1(p, sp, b):
    return pl.pallas_call(
        _combine1_body,
        grid=(_NB,),
        in_specs=[
            pl.BlockSpec((2, _BN, _D), lambda i: (0, i, 0)),
            pl.BlockSpec((_BN, _D), lambda i: (i, 0)),
            pl.BlockSpec((1, _D), lambda i: (0, 0)),
        ],
        out_specs=pl.BlockSpec((_BN, _D), lambda i: (i, 0)),
        out_shape=jax.ShapeDtypeStruct((_N, _D), jnp.float32),
    )(p, sp, b)


def _combine2_body(p_ref, sp_ref, b_ref, fcw_ref, fcb_ref, o_ref, acc_ref):
    i = pl.program_id(0)

    @pl.when(i == 0)
    def _():
        acc_ref[...] = jnp.zeros_like(acc_ref)

    h = jnp.maximum(p_ref[0] + p_ref[1] + sp_ref[...] + b_ref[...], 0.0)
    acc_ref[0:1] += jnp.sum(h, axis=0, keepdims=True)

    @pl.when(i == pl.num_programs(0) - 1)
    def _():
        hg = acc_ref[0:1] * (1.0 / _N)
        z = jnp.sum(hg * fcw_ref[...], keepdims=True) + fcb_ref[...]
        o_ref[...] = 1.0 / (1.0 + jnp.exp(-z))


def _combine2(p, sp, b, fcw_row, fcb):
    return pl.pallas_call(
        _combine2_body,
        grid=(_NB,),
        in_specs=[
            pl.BlockSpec((2, _BN, _D), lambda i: (0, i, 0)),
            pl.BlockSpec((_BN, _D), lambda i: (i, 0)),
            pl.BlockSpec((1, _D), lambda i: (0, 0)),
            pl.BlockSpec((1, _D), lambda i: (0, 0)),
            pl.BlockSpec((1, 1), lambda i: (0, 0)),
        ],
        out_specs=pl.BlockSpec((1, 1), lambda i: (0, 0)),
        out_shape=jax.ShapeDtypeStruct((1, 1), jnp.float32),
        scratch_shapes=[pltpu.VMEM((8, _D), jnp.float32)],
    )(p, sp, b, fcw_row, fcb)


def kernel(in_feat, edge_index, e_types, W1, Wself1, b1, W2, Wself2, b2,
           fc_w, fc_b):
    src = edge_index[0]
    dst = edge_index[1]
    pad = _EPAD - _E
    et_p = jnp.concatenate([e_types, jnp.zeros((pad,), jnp.int32)])
    src_p = jnp.concatenate([src, jnp.zeros((pad,), jnp.int32)])
    dst_p = jnp.concatenate([dst, jnp.full((pad,), _N, jnp.int32)])

    gidx = _gidx(et_p.reshape(-1, _CH), src_p.reshape(-1, _CH))
    gidx4 = gidx.reshape(_NC, _NS, _NSEC, _SECN, _CH)
    didx4 = dst_p.reshape(_NC, _NS, _NSEC, _SECN, _CH)
    zeros = jnp.zeros((_NPAD, _D), jnp.float32)

    def layer(x, W, Wself):
        xw = _xw(x, W)
        sp = _selfp(x, Wself)
        p = _sc_agg(xw.reshape(_R * _N, _D), gidx4, didx4, zeros)
        return p.reshape(_NC, _NPAD, _D), sp

    p1, sp1 = layer(in_feat, W1, Wself1)
    h1 = _combine1(p1, sp1, b1.reshape(1, _D))
    p2, sp2 = layer(h1, W2, Wself2)
    return _combine2(p2, sp2, b2.reshape(1, _D), fc_w.reshape(1, _D),
                     fc_b.reshape(1, 1))
